# baseline (speedup 1.0000x reference)
import functools

import jax
import jax.numpy as jnp
from jax import lax
from jax.experimental import pallas as pl
from jax.experimental.pallas import tpu as pltpu
from jax.experimental.pallas import tpu_sc as plsc

N = 10000
E = 320000
D = 128
H = 32

NW = 32
KB = 79
EPW = KB * 128
EPAD = NW * EPW
NPAD = 10112
RPT = NPAD // 16

ROWB = 2000
DEC_BR = 200


def _mm_body(x_ref, ws_ref, wn_ref, s_ref, p_ref):
    x = x_ref[...]
    s_ref[...] = jnp.dot(x, ws_ref[...], preferred_element_type=jnp.float32)
    p_ref[...] = jnp.dot(x, wn_ref[...], preferred_element_type=jnp.float32)


def _matmul2(x, ws, wn):
    return pl.pallas_call(
        _mm_body,
        grid=(N // ROWB,),
        in_specs=[pl.BlockSpec((ROWB, D), lambda i: (i, 0)),
                  pl.BlockSpec((D, H), lambda i: (0, 0)),
                  pl.BlockSpec((D, H), lambda i: (0, 0))],
        out_specs=[pl.BlockSpec((ROWB, H), lambda i: (i, 0))] * 2,
        out_shape=[jax.ShapeDtypeStruct((N, H), jnp.float32)] * 2,
    )(x, ws, wn)


def _sc_aggregate(table, srcw, dstw, zagg, zdeg, ones16, with_deg):
    mesh = plsc.VectorSubcoreMesh(core_axis_name="c", subcore_axis_name="s")

    out_type = [jax.ShapeDtypeStruct((NPAD, H), jnp.float32),
                jax.ShapeDtypeStruct((NPAD, H), jnp.float32)]
    scratch = [pltpu.VMEM((KB, 128), jnp.int32),
               pltpu.VMEM((KB, 128), jnp.int32),
               pltpu.VMEM((128, H), jnp.float32),
               pltpu.VMEM((128, H), jnp.float32),
               pltpu.VMEM((128, H), jnp.float32),
               pltpu.VMEM((128, H), jnp.float32),
               pltpu.VMEM_SHARED((NPAD, H), jnp.float32),
               pltpu.SemaphoreType.DMA,
               pltpu.SemaphoreType.DMA,
               pltpu.SemaphoreType.DMA,
               pltpu.SemaphoreType.DMA,
               pltpu.SemaphoreType.DMA,
               pltpu.SemaphoreType.DMA,
               pltpu.SemaphoreType.DMA,
               pltpu.SemaphoreType.DMA]
    if with_deg:
        out_type += [jax.ShapeDtypeStruct((NPAD, 16), jnp.float32),
                     jax.ShapeDtypeStruct((NPAD, 16), jnp.float32)]
        scratch += [pltpu.VMEM((128, 16), jnp.float32),
                    pltpu.VMEM_SHARED((NPAD, 16), jnp.float32)]

    def body(*refs):
        if with_deg:
            (tab_hbm, src_hbm, dst_hbm, zagg_hbm, zdeg_hbm, ones_hbm,
             agg0_hbm, agg1_hbm, deg0_hbm, deg1_hbm,
             src_v, dst_v, r0, r1, r2, r3, sh_agg,
             g0, g1, g2, g3, s0, s1_, s2, s3,
             ones_v, sh_deg) = refs
        else:
            (tab_hbm, src_hbm, dst_hbm, zagg_hbm,
             agg0_hbm, agg1_hbm,
             src_v, dst_v, r0, r1, r2, r3, sh_agg,
             g0, g1, g2, g3, s0, s1_, s2, s3) = refs
        rows = [r0, r1, r2, r3]
        sem_g = [g0, g1, g2, g3]
        sem_s = [s0, s1_, s2, s3]
        c = lax.axis_index("c")
        s = lax.axis_index("s")
        w = s * 2 + c

        @pl.when(s == 0)
        def _():
            pltpu.sync_copy(zagg_hbm, sh_agg)
            if with_deg:
                pltpu.sync_copy(zdeg_hbm, sh_deg)

        pltpu.sync_copy(src_hbm.at[w], src_v)
        pltpu.sync_copy(dst_hbm.at[w], dst_v)
        if with_deg:
            pltpu.sync_copy(ones_hbm, ones_v)
        plsc.subcore_barrier()

        def gather(j, b):
            pltpu.async_copy(tab_hbm.at[src_v.at[j]], rows[b], sem_g[b])

        def wait_g(j, b):
            pltpu.make_async_copy(
                tab_hbm.at[src_v.at[j]], rows[b], sem_g[b]).wait()

        def scatter(j, b):
            pltpu.async_copy(rows[b], sh_agg.at[dst_v.at[j]], sem_s[b],
                             add=True)
            if with_deg:
                pltpu.async_copy(ones_v, sh_deg.at[dst_v.at[j]], sem_s[b],
                                 add=True)

        def wait_s(j, b):
            pltpu.make_async_copy(
                rows[b], sh_agg.at[dst_v.at[j]], sem_s[b]).wait()
            if with_deg:
                pltpu.make_async_copy(
                    ones_v, sh_deg.at[dst_v.at[j]], sem_s[b]).wait()

        def slot(j, ws=True, g=True):
            b = j % 4
            wait_g(j, b)
            scatter(j, b)
            if ws:
                wait_s(j - 2, (j - 2) % 4)
            if g:
                gather(j + 2, (j + 2) % 4)

        gather(0, 0)
        gather(1, 1)
        slot(0, ws=False)
        slot(1, ws=False)

        def step(jj, carry):
            j0 = jj * 4 + 2
            for k in range(4):
                j = j0 + k
                b = (2 + k) % 4
                wait_g(j, b)
                scatter(j, b)
                wait_s(j - 2, (b + 2) % 4)
                gather(j + 2, (b + 2) % 4)
            return carry

        lax.fori_loop(0, (KB - 7) // 4, step, 0)
        slot(KB - 5)
        slot(KB - 4)
        slot(KB - 3)
        slot(KB - 2, g=False)
        slot(KB - 1, g=False)
        wait_s(KB - 2, (KB - 2) % 4)
        wait_s(KB - 1, (KB - 1) % 4)
        plsc.subcore_barrier()

        row0 = s * RPT

        @pl.when(c == 0)
        def _():
            pltpu.sync_copy(sh_agg.at[pl.ds(row0, RPT)],
                            agg0_hbm.at[pl.ds(row0, RPT)])
            if with_deg:
                pltpu.sync_copy(sh_deg.at[pl.ds(row0, RPT)],
                                deg0_hbm.at[pl.ds(row0, RPT)])

        @pl.when(c == 1)
        def _():
            pltpu.sync_copy(sh_agg.at[pl.ds(row0, RPT)],
                            agg1_hbm.at[pl.ds(row0, RPT)])
            if with_deg:
                pltpu.sync_copy(sh_deg.at[pl.ds(row0, RPT)],
                                deg1_hbm.at[pl.ds(row0, RPT)])

    k = functools.partial(
        pl.kernel, mesh=mesh, out_type=out_type, scratch_types=scratch,
        compiler_params=pltpu.CompilerParams(use_tc_tiling_on_sc=False),
    )(body)
    if with_deg:
        return k(table, srcw, dstw, zagg, zdeg, ones16)
    return k(table, srcw, dstw, zagg)


def _h_body(s1_ref, a0_ref, a1_ref, d0_ref, d1_ref, b_ref, h_ref):
    deg = d0_ref[:, 0:1] + d1_ref[:, 0:1]
    recip = 1.0 / jnp.maximum(deg, 1.0)
    agg = a0_ref[...] + a1_ref[...]
    h_ref[...] = jnp.maximum(s1_ref[...] + agg * recip + b_ref[...], 0.0)


def _h_combine(s1, agg0, agg1, deg0, deg1, b1):
    row2 = lambda i: (i, 0)
    return pl.pallas_call(
        _h_body,
        grid=(N // ROWB,),
        in_specs=[pl.BlockSpec((ROWB, H), row2),
                  pl.BlockSpec((ROWB, H), row2),
                  pl.BlockSpec((ROWB, H), row2),
                  pl.BlockSpec((ROWB, 16), row2),
                  pl.BlockSpec((ROWB, 16), row2),
                  pl.BlockSpec((1, H), lambda i: (0, 0))],
        out_specs=pl.BlockSpec((ROWB, H), row2),
        out_shape=jax.ShapeDtypeStruct((N, H), jnp.float32),
    )(s1, agg0, agg1, deg0, deg1, b1.reshape(1, H))


def _mll_body(h_ref, a0_ref, a1_ref, d0_ref, d1_ref,
              wsm_ref, wnm_ref, bm_ref, wsl_ref, wnl_ref, bl_ref, eps_ref,
              mu_ref, ls_ref, z_ref):
    deg = d0_ref[:, 0:1] + d1_ref[:, 0:1]
    recip = 1.0 / jnp.maximum(deg, 1.0)
    m = (a0_ref[...] + a1_ref[...]) * recip
    h = h_ref[...]
    mu = (jnp.dot(h, wsm_ref[...], preferred_element_type=jnp.float32)
          + jnp.dot(m, wnm_ref[...], preferred_element_type=jnp.float32)
          + bm_ref[...])
    ls = (jnp.dot(h, wsl_ref[...], preferred_element_type=jnp.float32)
          + jnp.dot(m, wnl_ref[...], preferred_element_type=jnp.float32)
          + bl_ref[...])
    mu_ref[...] = mu
    ls_ref[...] = ls
    z_ref[...] = mu + eps_ref[...] * jnp.exp(ls)


def _mu_logstd_z(h, agg0, agg1, deg0, deg1, Wsm, Wnm, bm, Wsl, Wnl, bl, eps):
    row = lambda i: (i, 0)
    fix = lambda i: (0, 0)
    return pl.pallas_call(
        _mll_body,
        grid=(N // ROWB,),
        in_specs=[pl.BlockSpec((ROWB, H), row),
                  pl.BlockSpec((ROWB, H), row),
                  pl.BlockSpec((ROWB, H), row),
                  pl.BlockSpec((ROWB, 16), row),
                  pl.BlockSpec((ROWB, 16), row),
                  pl.BlockSpec((H, H), fix),
                  pl.BlockSpec((H, H), fix),
                  pl.BlockSpec((1, H), fix),
                  pl.BlockSpec((H, H), fix),
                  pl.BlockSpec((H, H), fix),
                  pl.BlockSpec((1, H), fix),
                  pl.BlockSpec((ROWB, H), row)],
        out_specs=[pl.BlockSpec((ROWB, H), row)] * 3,
        out_shape=[jax.ShapeDtypeStruct((N, H), jnp.float32)] * 3,
    )(h, agg0, agg1, deg0, deg1, Wsm, Wnm, bm.reshape(1, H),
      Wsl, Wnl, bl.reshape(1, H), eps)


def _dec_body(zi_ref, zj_ref, o_ref):
    d = jnp.dot(zi_ref[...], zj_ref[...], preferred_element_type=jnp.float32)
    o_ref[...] = 0.5 * jnp.tanh(0.5 * d) + 0.5


def _decode(z, zT):
    return pl.pallas_call(
        _dec_body,
        grid=(N // DEC_BR,),
        in_specs=[pl.BlockSpec((DEC_BR, H), lambda i: (i, 0)),
                  pl.BlockSpec((H, N), lambda i: (0, 0))],
        out_specs=pl.BlockSpec((DEC_BR, N), lambda i: (i, 0)),
        out_shape=jax.ShapeDtypeStruct((N, N), jnp.float32),
    )(z, zT)


def kernel(n_feats, edge_index, Ws1, Wn1, b1, Wsm, Wnm, bm, Wsl, Wnl, bl):
    src = edge_index[0]
    dst = edge_index[1]
    pad = EPAD - E
    srcw = jnp.concatenate(
        [src, jnp.zeros((pad,), jnp.int32)]).reshape(NW, KB, 128)
    dstw = jnp.concatenate(
        [dst, jnp.full((pad,), N, jnp.int32)]).reshape(NW, KB, 128)
    zagg = jnp.zeros((NPAD, H), jnp.float32)
    zdeg = jnp.zeros((NPAD, 16), jnp.float32)
    ones16 = jnp.ones((128, 16), jnp.float32)

    s1, p = _matmul2(n_feats, Ws1, Wn1)

    agg0, agg1, deg0, deg1 = _sc_aggregate(
        p, srcw, dstw, zagg, zdeg, ones16, True)
    h = _h_combine(s1, agg0, agg1, deg0, deg1, b1)

    ah0, ah1 = _sc_aggregate(h, srcw, dstw, zagg, zdeg, ones16, False)

    eps = jax.random.normal(jax.random.key(42), (N, H), dtype=jnp.float32)
    mu, logstd, z = _mu_logstd_z(
        h, ah0, ah1, deg0, deg1, Wsm, Wnm, bm, Wsl, Wnl, bl, eps)

    adj = _decode(z, z.T)
    return adj, mu, logstd


# speedup vs baseline: 1.0150x; 1.0150x over previous
import functools

import jax
import jax.numpy as jnp
from jax import lax
from jax.experimental import pallas as pl
from jax.experimental.pallas import tpu as pltpu
from jax.experimental.pallas import tpu_sc as plsc

N = 10000
E = 320000
D = 128
H = 32

NW = 32
KB = 79
EPW = KB * 128
EPAD = NW * EPW
NPAD = 10112
RPT = NPAD // 16

ROWB = 2000
DEC_BR = 200


def _mm_body(x_ref, ws_ref, wn_ref, s_ref, p_ref):
    x = x_ref[...]
    s_ref[...] = jnp.dot(x, ws_ref[...], preferred_element_type=jnp.float32)
    p_ref[...] = jnp.dot(x, wn_ref[...], preferred_element_type=jnp.float32)


def _matmul2(x, ws, wn):
    return pl.pallas_call(
        _mm_body,
        grid=(N // ROWB,),
        in_specs=[pl.BlockSpec((ROWB, D), lambda i: (i, 0)),
                  pl.BlockSpec((D, H), lambda i: (0, 0)),
                  pl.BlockSpec((D, H), lambda i: (0, 0))],
        out_specs=[pl.BlockSpec((ROWB, H), lambda i: (i, 0))] * 2,
        out_shape=[jax.ShapeDtypeStruct((N, H), jnp.float32)] * 2,
    )(x, ws, wn)


def _sc_aggregate(table, srcw, dstw, zagg, zdeg, ones16, with_deg):
    mesh = plsc.VectorSubcoreMesh(core_axis_name="c", subcore_axis_name="s")

    out_type = [jax.ShapeDtypeStruct((NPAD, H), jnp.float32),
                jax.ShapeDtypeStruct((NPAD, H), jnp.float32)]
    scratch = [pltpu.VMEM((KB, 128), jnp.int32),
               pltpu.VMEM((KB, 128), jnp.int32),
               pltpu.VMEM((128, H), jnp.float32),
               pltpu.VMEM((128, H), jnp.float32),
               pltpu.VMEM((128, H), jnp.float32),
               pltpu.VMEM_SHARED((NPAD, H), jnp.float32),
               pltpu.SemaphoreType.DMA,
               pltpu.SemaphoreType.DMA,
               pltpu.SemaphoreType.DMA,
               pltpu.SemaphoreType.DMA,
               pltpu.SemaphoreType.DMA,
               pltpu.SemaphoreType.DMA]
    if with_deg:
        out_type += [jax.ShapeDtypeStruct((NPAD, 16), jnp.float32),
                     jax.ShapeDtypeStruct((NPAD, 16), jnp.float32)]
        scratch += [pltpu.VMEM((128, 16), jnp.float32),
                    pltpu.VMEM_SHARED((NPAD, 16), jnp.float32)]

    def body(*refs):
        if with_deg:
            (tab_hbm, src_hbm, dst_hbm, zagg_hbm, zdeg_hbm, ones_hbm,
             agg0_hbm, agg1_hbm, deg0_hbm, deg1_hbm,
             src_v, dst_v, r0, r1, r2, sh_agg, g0, g1, g2, s0, s1_, s2,
             ones_v, sh_deg) = refs
        else:
            (tab_hbm, src_hbm, dst_hbm, zagg_hbm,
             agg0_hbm, agg1_hbm,
             src_v, dst_v, r0, r1, r2, sh_agg, g0, g1, g2, s0, s1_, s2) = refs
        rows = [r0, r1, r2]
        sem_g = [g0, g1, g2]
        sem_s = [s0, s1_, s2]
        c = lax.axis_index("c")
        s = lax.axis_index("s")
        w = s * 2 + c

        @pl.when(s == 0)
        def _():
            pltpu.sync_copy(zagg_hbm, sh_agg)
            if with_deg:
                pltpu.sync_copy(zdeg_hbm, sh_deg)

        pltpu.sync_copy(src_hbm.at[w], src_v)
        pltpu.sync_copy(dst_hbm.at[w], dst_v)
        if with_deg:
            pltpu.sync_copy(ones_hbm, ones_v)
        plsc.subcore_barrier()

        def gather(j, b):
            pltpu.async_copy(tab_hbm.at[src_v.at[j]], rows[b], sem_g[b])

        def wait_g(j, b):
            pltpu.make_async_copy(
                tab_hbm.at[src_v.at[j]], rows[b], sem_g[b]).wait()

        def scatter(j, b):
            pltpu.async_copy(rows[b], sh_agg.at[dst_v.at[j]], sem_s[b],
                             add=True)
            if with_deg:
                pltpu.async_copy(ones_v, sh_deg.at[dst_v.at[j]], sem_s[b],
                                 add=True)

        def wait_s(j, b):
            pltpu.make_async_copy(
                rows[b], sh_agg.at[dst_v.at[j]], sem_s[b]).wait()
            if with_deg:
                pltpu.make_async_copy(
                    ones_v, sh_deg.at[dst_v.at[j]], sem_s[b]).wait()

        def slot(j, ws=True, g=True):
            b = j % 3
            wait_g(j, b)
            scatter(j, b)
            if ws:
                wait_s(j - 1, (j - 1) % 3)
            if g:
                gather(j + 2, (j + 2) % 3)

        gather(0, 0)
        gather(1, 1)
        slot(0, ws=False)
        slot(1)

        def step(jj, carry):
            j0 = jj * 3 + 2
            for k in range(3):
                j = j0 + k
                b = (2 + k) % 3
                wait_g(j, b)
                scatter(j, b)
                wait_s(j - 1, (b + 2) % 3)
                gather(j + 2, (b + 2) % 3)
            return carry

        lax.fori_loop(0, (KB - 4) // 3, step, 0)
        slot(KB - 2, g=False)
        slot(KB - 1, g=False)
        wait_s(KB - 1, (KB - 1) % 3)
        plsc.subcore_barrier()

        row0 = s * RPT

        @pl.when(c == 0)
        def _():
            pltpu.sync_copy(sh_agg.at[pl.ds(row0, RPT)],
                            agg0_hbm.at[pl.ds(row0, RPT)])
            if with_deg:
                pltpu.sync_copy(sh_deg.at[pl.ds(row0, RPT)],
                                deg0_hbm.at[pl.ds(row0, RPT)])

        @pl.when(c == 1)
        def _():
            pltpu.sync_copy(sh_agg.at[pl.ds(row0, RPT)],
                            agg1_hbm.at[pl.ds(row0, RPT)])
            if with_deg:
                pltpu.sync_copy(sh_deg.at[pl.ds(row0, RPT)],
                                deg1_hbm.at[pl.ds(row0, RPT)])

    k = functools.partial(
        pl.kernel, mesh=mesh, out_type=out_type, scratch_types=scratch,
        compiler_params=pltpu.CompilerParams(use_tc_tiling_on_sc=False),
    )(body)
    if with_deg:
        return k(table, srcw, dstw, zagg, zdeg, ones16)
    return k(table, srcw, dstw, zagg)


def _h_body(s1_ref, a0_ref, a1_ref, d0_ref, d1_ref, b_ref, h_ref):
    deg = d0_ref[:, 0:1] + d1_ref[:, 0:1]
    recip = 1.0 / jnp.maximum(deg, 1.0)
    agg = a0_ref[...] + a1_ref[...]
    h_ref[...] = jnp.maximum(s1_ref[...] + agg * recip + b_ref[...], 0.0)


def _h_combine(s1, agg0, agg1, deg0, deg1, b1):
    row2 = lambda i: (i, 0)
    return pl.pallas_call(
        _h_body,
        grid=(N // ROWB,),
        in_specs=[pl.BlockSpec((ROWB, H), row2),
                  pl.BlockSpec((ROWB, H), row2),
                  pl.BlockSpec((ROWB, H), row2),
                  pl.BlockSpec((ROWB, 16), row2),
                  pl.BlockSpec((ROWB, 16), row2),
                  pl.BlockSpec((1, H), lambda i: (0, 0))],
        out_specs=pl.BlockSpec((ROWB, H), row2),
        out_shape=jax.ShapeDtypeStruct((N, H), jnp.float32),
    )(s1, agg0, agg1, deg0, deg1, b1.reshape(1, H))


def _mll_body(h_ref, a0_ref, a1_ref, d0_ref, d1_ref,
              wsm_ref, wnm_ref, bm_ref, wsl_ref, wnl_ref, bl_ref, eps_ref,
              mu_ref, ls_ref, z_ref):
    deg = d0_ref[:, 0:1] + d1_ref[:, 0:1]
    recip = 1.0 / jnp.maximum(deg, 1.0)
    m = (a0_ref[...] + a1_ref[...]) * recip
    h = h_ref[...]
    mu = (jnp.dot(h, wsm_ref[...], preferred_element_type=jnp.float32)
          + jnp.dot(m, wnm_ref[...], preferred_element_type=jnp.float32)
          + bm_ref[...])
    ls = (jnp.dot(h, wsl_ref[...], preferred_element_type=jnp.float32)
          + jnp.dot(m, wnl_ref[...], preferred_element_type=jnp.float32)
          + bl_ref[...])
    mu_ref[...] = mu
    ls_ref[...] = ls
    z_ref[...] = mu + eps_ref[...] * jnp.exp(ls)


def _mu_logstd_z(h, agg0, agg1, deg0, deg1, Wsm, Wnm, bm, Wsl, Wnl, bl, eps):
    row = lambda i: (i, 0)
    fix = lambda i: (0, 0)
    return pl.pallas_call(
        _mll_body,
        grid=(N // ROWB,),
        in_specs=[pl.BlockSpec((ROWB, H), row),
                  pl.BlockSpec((ROWB, H), row),
                  pl.BlockSpec((ROWB, H), row),
                  pl.BlockSpec((ROWB, 16), row),
                  pl.BlockSpec((ROWB, 16), row),
                  pl.BlockSpec((H, H), fix),
                  pl.BlockSpec((H, H), fix),
                  pl.BlockSpec((1, H), fix),
                  pl.BlockSpec((H, H), fix),
                  pl.BlockSpec((H, H), fix),
                  pl.BlockSpec((1, H), fix),
                  pl.BlockSpec((ROWB, H), row)],
        out_specs=[pl.BlockSpec((ROWB, H), row)] * 3,
        out_shape=[jax.ShapeDtypeStruct((N, H), jnp.float32)] * 3,
    )(h, agg0, agg1, deg0, deg1, Wsm, Wnm, bm.reshape(1, H),
      Wsl, Wnl, bl.reshape(1, H), eps)


def _dec_body(zi_ref, zj_ref, o_ref):
    d = jnp.dot(zi_ref[...], zj_ref[...], preferred_element_type=jnp.float32)
    o_ref[...] = 0.5 * jnp.tanh(0.5 * d) + 0.5


def _decode(z, zT):
    return pl.pallas_call(
        _dec_body,
        grid=(N // DEC_BR,),
        in_specs=[pl.BlockSpec((DEC_BR, H), lambda i: (i, 0)),
                  pl.BlockSpec((H, N), lambda i: (0, 0))],
        out_specs=pl.BlockSpec((DEC_BR, N), lambda i: (i, 0)),
        out_shape=jax.ShapeDtypeStruct((N, N), jnp.float32),
    )(z, zT)


def kernel(n_feats, edge_index, Ws1, Wn1, b1, Wsm, Wnm, bm, Wsl, Wnl, bl):
    src = edge_index[0]
    dst = edge_index[1]
    pad = EPAD - E
    srcw = jnp.concatenate(
        [src, jnp.zeros((pad,), jnp.int32)]).reshape(NW, KB, 128)
    dstw = jnp.concatenate(
        [dst, jnp.full((pad,), N, jnp.int32)]).reshape(NW, KB, 128)
    zagg = jnp.zeros((NPAD, H), jnp.float32)
    zdeg = jnp.zeros((NPAD, 16), jnp.float32)
    ones16 = jnp.ones((128, 16), jnp.float32)

    s1, p = _matmul2(n_feats, Ws1, Wn1)

    agg0, agg1, deg0, deg1 = _sc_aggregate(
        p, srcw, dstw, zagg, zdeg, ones16, True)
    h = _h_combine(s1, agg0, agg1, deg0, deg1, b1)

    ah0, ah1 = _sc_aggregate(h, srcw, dstw, zagg, zdeg, ones16, False)

    eps = jax.random.normal(jax.random.key(42), (N, H), dtype=jnp.float32)
    mu, logstd, z = _mu_logstd_z(
        h, ah0, ah1, deg0, deg1, Wsm, Wnm, bm, Wsl, Wnl, bl, eps)

    adj = _decode(z, z.T)
    return adj, mu, logstd
